# baseline (speedup 1.0000x reference)
import jax
import jax.numpy as jnp
from jax import lax
from jax.experimental import pallas as pl
from jax.experimental.pallas import tpu as pltpu
from jax.experimental.pallas import tpu_sc as plsc

VOCAB = 100000
DIM = 64
BATCH = 4096
HIST = 200

NUM_CORES = 2
NUM_SUBCORES = 16
NUM_WORKERS = NUM_CORES * NUM_SUBCORES
BLK = BATCH // NUM_WORKERS
NSLAB = HIST * NUM_WORKERS
SLABS_PER_TC_BLK = 8


def _gather_body(table_hbm, idxp_hbm, out_hbm,
                 idx_v, rows0, rows1, sem_g0, sem_g1, sem_s0, sem_s1):
    wid = lax.axis_index("s") * NUM_CORES + lax.axis_index("c")
    rows_v = (rows0, rows1)
    sem_g = (sem_g0, sem_g1)
    sem_s = (sem_s0, sem_s1)

    pltpu.sync_copy(idxp_hbm.at[:, pl.ds(wid * BLK, BLK)], idx_v)

    pltpu.async_copy(table_hbm.at[idx_v.at[0]], rows_v[0], sem_g[0])

    def step(h, carry):
        par = lax.rem(h, 2)

        def run(b, p):
            @pl.when(h >= 2)
            def _():
                pltpu.make_async_copy(rows_v[b], out_hbm.at[0],
                                      sem_s[b]).wait()

            @pl.when(h < HIST)
            def _():
                pltpu.async_copy(table_hbm.at[idx_v.at[h]], rows_v[b],
                                 sem_g[b])

            pltpu.make_async_copy(table_hbm.at[idx_v.at[0]], rows_v[p],
                                  sem_g[p]).wait()
            pltpu.async_copy(rows_v[p],
                             out_hbm.at[(h - 1) * NUM_WORKERS + wid],
                             sem_s[p])

        @pl.when(par == 0)
        def _():
            run(0, 1)

        @pl.when(par == 1)
        def _():
            run(1, 0)

        return carry

    lax.fori_loop(1, HIST + 1, step, 0)

    pltpu.make_async_copy(rows_v[1], out_hbm.at[0], sem_s[1]).wait()


def _tc_format(x):
    def body(x_ref, o_ref):
        for k in range(SLABS_PER_TC_BLK):
            xs = x_ref[pl.ds(k * 64, 64), :]
            t = xs.T
            y = jnp.concatenate([t[0:64, :], t[64:128, :]], axis=1)
            o_ref[0, :, k, :, :] = y.reshape(8, 8, 128)

    return pl.pallas_call(
        body,
        grid=(NSLAB // SLABS_PER_TC_BLK,),
        in_specs=[pl.BlockSpec((64 * SLABS_PER_TC_BLK, 128),
                               lambda u: (u, 0))],
        out_specs=pl.BlockSpec((1, 8, SLABS_PER_TC_BLK, 8, 128),
                               lambda u: (u // 4, 0, u % 4, 0, 0)),
        out_shape=jax.ShapeDtypeStruct((HIST, 8, NUM_WORKERS, 8, 128),
                                       jnp.float32),
    )(x)


@jax.jit
def _embed(embeddings, idxp):
    mesh = plsc.VectorSubcoreMesh(core_axis_name="c", subcore_axis_name="s")
    f = pl.kernel(
        _gather_body,
        mesh=mesh,
        compiler_params=pltpu.CompilerParams(use_tc_tiling_on_sc=False),
        out_type=jax.ShapeDtypeStruct((NSLAB, BLK, DIM), jnp.float32),
        scratch_types=[
            pltpu.VMEM((HIST, BLK), jnp.int32),
            pltpu.VMEM((BLK, DIM), jnp.float32),
            pltpu.VMEM((BLK, DIM), jnp.float32),
            pltpu.SemaphoreType.DMA,
            pltpu.SemaphoreType.DMA,
            pltpu.SemaphoreType.DMA,
            pltpu.SemaphoreType.DMA,
        ],
    )
    g = f(embeddings, idxp)
    out5 = _tc_format(g.reshape(NSLAB * BLK // 2, 2 * DIM))
    return out5


def kernel(inputs, embeddings):
    idxp = (inputs.T.reshape(HIST, NUM_WORKERS, 2, 64)
            .transpose(0, 1, 3, 2).reshape(HIST, BATCH))
    out5 = _embed(embeddings, idxp)
    return out5.transpose(2, 4, 0, 1, 3).reshape(BATCH, HIST, DIM)


# speedup vs baseline: 1.7138x; 1.7138x over previous
import jax
import jax.numpy as jnp
from jax import lax
from jax.experimental import pallas as pl
from jax.experimental.pallas import tpu as pltpu
from jax.experimental.pallas import tpu_sc as plsc

VOCAB = 100000
DIM = 64
BATCH = 4096
HIST = 200

NUM_CORES = 2
NUM_SUBCORES = 16
NUM_WORKERS = NUM_CORES * NUM_SUBCORES
BLK = BATCH // NUM_WORKERS
NPAIR = NUM_WORKERS // 2
NSLAB2 = HIST * NPAIR


def _gather_body(table_hbm, idxp_hbm, out_hbm,
                 idx_v, rows0, rows1, sem_g0, sem_g1, sem_s0, sem_s1):
    wid = lax.axis_index("s") * NUM_CORES + lax.axis_index("c")
    pair = wid // 2
    member = lax.rem(wid, 2)
    rows_v = (rows0, rows1)
    sem_g = (sem_g0, sem_g1)
    sem_s = (sem_s0, sem_s1)

    pltpu.sync_copy(idxp_hbm.at[:, pl.ds(wid * BLK, BLK)], idx_v)

    pltpu.async_copy(table_hbm.at[idx_v.at[0]], rows_v[0], sem_g[0])

    def step(h, carry):
        par = lax.rem(h, 2)

        def run(b, p):
            @pl.when(h >= 2)
            def _():
                pltpu.make_async_copy(rows_v[b], out_hbm.at[0, 0],
                                      sem_s[b]).wait()

            @pl.when(h < HIST)
            def _():
                pltpu.async_copy(table_hbm.at[idx_v.at[h]], rows_v[b],
                                 sem_g[b])

            pltpu.make_async_copy(table_hbm.at[idx_v.at[0]], rows_v[p],
                                  sem_g[p]).wait()
            pltpu.async_copy(rows_v[p],
                             out_hbm.at[(h - 1) * NPAIR + pair, member],
                             sem_s[p])

        @pl.when(par == 0)
        def _():
            run(0, 1)

        @pl.when(par == 1)
        def _():
            run(1, 0)

        return carry

    lax.fori_loop(1, HIST + 1, step, 0)

    pltpu.make_async_copy(rows_v[1], out_hbm.at[0, 0], sem_s[1]).wait()


def _tc_format(x):
    def body(x_ref, o_ref):
        for k in range(NPAIR):
            xs = x_ref[pl.ds(k * 128, 128), :]
            t = xs.T
            y = t.reshape(2, 8, 8, 128).transpose(1, 0, 2, 3)
            o_ref[0, :, pl.ds(2 * k, 2), :, :] = y

    return pl.pallas_call(
        body,
        grid=(HIST,),
        in_specs=[pl.BlockSpec((NPAIR * 128, 128), lambda u: (u, 0))],
        out_specs=pl.BlockSpec((1, 8, NUM_WORKERS, 8, 128),
                               lambda u: (u, 0, 0, 0, 0)),
        out_shape=jax.ShapeDtypeStruct((HIST, 8, NUM_WORKERS, 8, 128),
                                       jnp.float32),
    )(x)


@jax.jit
def _embed(embeddings, idxp):
    mesh = plsc.VectorSubcoreMesh(core_axis_name="c", subcore_axis_name="s")
    f = pl.kernel(
        _gather_body,
        mesh=mesh,
        compiler_params=pltpu.CompilerParams(use_tc_tiling_on_sc=False),
        out_type=jax.ShapeDtypeStruct((NSLAB2, 2, BLK, DIM), jnp.float32),
        scratch_types=[
            pltpu.VMEM((HIST, BLK), jnp.int32),
            pltpu.VMEM((BLK, DIM), jnp.float32),
            pltpu.VMEM((BLK, DIM), jnp.float32),
            pltpu.SemaphoreType.DMA,
            pltpu.SemaphoreType.DMA,
            pltpu.SemaphoreType.DMA,
            pltpu.SemaphoreType.DMA,
        ],
    )
    g = f(embeddings, idxp)
    out5 = _tc_format(g.reshape(NSLAB2 * BLK, 2 * DIM))
    return out5


def kernel(inputs, embeddings):
    idxp = (inputs.T.reshape(HIST, NPAIR, 2, 128)
            .transpose(0, 1, 3, 2).reshape(HIST, BATCH))
    out5 = _embed(embeddings, idxp)
    return out5.transpose(2, 4, 0, 1, 3).reshape(BATCH, HIST, DIM)


# speedup vs baseline: 1.9864x; 1.1590x over previous
import jax
import jax.numpy as jnp
from jax import lax
from jax.experimental import pallas as pl
from jax.experimental.pallas import tpu as pltpu
from jax.experimental.pallas import tpu_sc as plsc

VOCAB = 100000
DIM = 64
BATCH = 4096
HIST = 200

NUM_CORES = 2
NUM_SUBCORES = 16
NUM_WORKERS = NUM_CORES * NUM_SUBCORES
BLK = BATCH // NUM_WORKERS
NPAIR = NUM_WORKERS // 2
NCHUNK = 4
HCHUNK = HIST // NCHUNK
NSLAB2 = HCHUNK * NPAIR


def _gather_body(table_hbm, idxp_hbm, out_hbm,
                 idx_v, rows0, rows1, sem_g0, sem_g1, sem_s0, sem_s1,
                 *, h0):
    wid = lax.axis_index("s") * NUM_CORES + lax.axis_index("c")
    pair = wid // 2
    member = lax.rem(wid, 2)
    rows_v = (rows0, rows1)
    sem_g = (sem_g0, sem_g1)
    sem_s = (sem_s0, sem_s1)

    pltpu.sync_copy(idxp_hbm.at[pl.ds(h0, HCHUNK), pl.ds(wid * BLK, BLK)],
                    idx_v)

    pltpu.async_copy(table_hbm.at[idx_v.at[0]], rows_v[0], sem_g[0])

    def step(h, carry):
        par = lax.rem(h, 2)

        def run(b, p):
            @pl.when(h >= 2)
            def _():
                pltpu.make_async_copy(rows_v[b], out_hbm.at[0, 0],
                                      sem_s[b]).wait()

            @pl.when(h < HCHUNK)
            def _():
                pltpu.async_copy(table_hbm.at[idx_v.at[h]], rows_v[b],
                                 sem_g[b])

            pltpu.make_async_copy(table_hbm.at[idx_v.at[0]], rows_v[p],
                                  sem_g[p]).wait()
            pltpu.async_copy(rows_v[p],
                             out_hbm.at[(h - 1) * NPAIR + pair, member],
                             sem_s[p])

        @pl.when(par == 0)
        def _():
            run(0, 1)

        @pl.when(par == 1)
        def _():
            run(1, 0)

        return carry

    lax.fori_loop(1, HCHUNK + 1, step, 0)

    pltpu.make_async_copy(rows_v[1], out_hbm.at[0, 0], sem_s[1]).wait()


def _make_gather(h0):
    import functools
    mesh = plsc.VectorSubcoreMesh(core_axis_name="c", subcore_axis_name="s")
    return pl.kernel(
        functools.partial(_gather_body, h0=h0),
        mesh=mesh,
        compiler_params=pltpu.CompilerParams(use_tc_tiling_on_sc=False),
        out_type=jax.ShapeDtypeStruct((NSLAB2, 2, BLK, DIM), jnp.float32),
        scratch_types=[
            pltpu.VMEM((HCHUNK, BLK), jnp.int32),
            pltpu.VMEM((BLK, DIM), jnp.float32),
            pltpu.VMEM((BLK, DIM), jnp.float32),
            pltpu.SemaphoreType.DMA,
            pltpu.SemaphoreType.DMA,
            pltpu.SemaphoreType.DMA,
            pltpu.SemaphoreType.DMA,
        ],
    )


def _tc_body(x_ref, o_ref):
    for k in range(NPAIR):
        xs = x_ref[pl.ds(k * 128, 128), :]
        t = xs.T
        y = t.reshape(2, 8, 8, 128).transpose(1, 0, 2, 3)
        o_ref[0, :, pl.ds(2 * k, 2), :, :] = y


def _tc_format_first(x):
    return pl.pallas_call(
        _tc_body,
        grid=(HCHUNK,),
        in_specs=[pl.BlockSpec((NPAIR * 128, 128), lambda u: (u, 0))],
        out_specs=pl.BlockSpec((1, 8, NUM_WORKERS, 8, 128),
                               lambda u: (u, 0, 0, 0, 0)),
        out_shape=jax.ShapeDtypeStruct((HIST, 8, NUM_WORKERS, 8, 128),
                                       jnp.float32),
    )(x)


def _tc_format_chunk(acc, x, k):
    def body(acc_ref, x_ref, o_ref):
        _tc_body(x_ref, o_ref)

    return pl.pallas_call(
        body,
        grid=(HCHUNK,),
        in_specs=[pl.BlockSpec(memory_space=pl.ANY),
                  pl.BlockSpec((NPAIR * 128, 128), lambda u: (u, 0))],
        out_specs=pl.BlockSpec((1, 8, NUM_WORKERS, 8, 128),
                               lambda u, k=k: (k * HCHUNK + u, 0, 0, 0, 0)),
        out_shape=jax.ShapeDtypeStruct((HIST, 8, NUM_WORKERS, 8, 128),
                                       jnp.float32),
        input_output_aliases={0: 0},
    )(acc, x)


@jax.jit
def _embed(embeddings, idxp):
    gathers = [_make_gather(k * HCHUNK)(embeddings, idxp)
               for k in range(NCHUNK)]
    acc = _tc_format_first(gathers[0].reshape(NSLAB2 * BLK, 2 * DIM))
    for k in range(1, NCHUNK):
        acc = _tc_format_chunk(acc, gathers[k].reshape(NSLAB2 * BLK, 2 * DIM),
                               k)
    return acc


def kernel(inputs, embeddings):
    idxp = (inputs.reshape(NPAIR, 2, BLK, HIST)
            .transpose(3, 0, 2, 1).reshape(HIST, BATCH))
    out5 = _embed(embeddings, idxp)
    return out5.transpose(2, 4, 0, 1, 3).reshape(BATCH, HIST, DIM)
